# baseline (speedup 1.0000x reference)
import functools

import jax
import jax.numpy as jnp
from jax import lax
from jax.experimental import pallas as pl
from jax.experimental.pallas import tpu as pltpu
from jax.experimental.pallas import tpu_sc as plsc

MAXLEN = 200
EMBED = 64
BATCH = 1024
NC = 2
NS = 16
NW = NC * NS
ROWS = BATCH * MAXLEN
R_PER_W = ROWS // NW
CHUNK = 128
NCHUNK = R_PER_W // CHUNK
NBUF = 6
LOOKAHEAD = 4


@functools.partial(
    pl.kernel,
    out_type=jax.ShapeDtypeStruct((ROWS, EMBED), jnp.float32),
    mesh=plsc.VectorSubcoreMesh(core_axis_name="c", subcore_axis_name="s"),
    compiler_params=pltpu.CompilerParams(use_tc_tiling_on_sc=False),
    scratch_types=[
        pltpu.VMEM((NCHUNK, CHUNK), jnp.int32),
        pltpu.VMEM((2 * MAXLEN, EMBED), jnp.float32),
        pltpu.VMEM((NBUF, CHUNK, EMBED), jnp.float32),
        pltpu.SemaphoreType.DMA,
        pltpu.SemaphoreType.DMA,
    ],
)
def _embed_kernel(x_hbm, tok_hbm, pos2_hbm, out_hbm, idx_v, pos_v, buf_v,
                  gsem, ssem):
    wid = lax.axis_index("s") * NC + lax.axis_index("c")
    base = wid * R_PER_W
    pltpu.sync_copy(x_hbm.at[pl.ds(wid * NCHUNK, NCHUNK)], idx_v)
    pltpu.sync_copy(pos2_hbm, pos_v)

    def start_gather(c):
        return pltpu.async_copy(tok_hbm.at[idx_v.at[c]],
                                buf_v.at[c % NBUF], gsem)

    gcp, scp = {}, {}
    for c in range(LOOKAHEAD):
        gcp[c] = start_gather(c)
    for c in range(NCHUNK):
        nc = c + LOOKAHEAD
        if nc < NCHUNK:
            oc = nc - NBUF
            if oc >= 0:
                scp.pop(oc).wait()
            gcp[nc] = start_gather(nc)
        gcp.pop(c).wait()
        k = c % NBUF
        p0 = (c * CHUNK) % MAXLEN

        def add_body(r, _, k=k, p0=p0):
            for c4 in range(EMBED // 16):
                sl = pl.ds(c4 * 16, 16)
                plsc.addupdate(buf_v.at[k, r, sl], pos_v[p0 + r, sl])
            return 0

        lax.fori_loop(0, CHUNK, add_body, 0, unroll=4)
        scp[c] = pltpu.async_copy(buf_v.at[k],
                                  out_hbm.at[pl.ds(base + c * CHUNK, CHUNK)],
                                  ssem)
    for c in sorted(scp):
        scp[c].wait()


def kernel(x, token_table, pos_table):
    x2 = x.astype(jnp.int32).reshape(ROWS // CHUNK, CHUNK)
    pos2 = jnp.concatenate([pos_table, pos_table], axis=0)
    out = _embed_kernel(x2, token_table, pos2)
    return out.reshape(BATCH, MAXLEN, EMBED)


# speedup vs baseline: 1.0144x; 1.0144x over previous
import functools

import jax
import jax.numpy as jnp
from jax import lax
from jax.experimental import pallas as pl
from jax.experimental.pallas import tpu as pltpu
from jax.experimental.pallas import tpu_sc as plsc

MAXLEN = 200
EMBED = 64
BATCH = 1024
NC = 2
NS = 16
NW = NC * NS
B_PER_W = BATCH // NW
IDX_MINOR = 100
GATHERS_PER_ROW = MAXLEN // IDX_MINOR
NBUF = 6
LOOKAHEAD = 3


@functools.partial(
    pl.kernel,
    out_type=jax.ShapeDtypeStruct((BATCH * MAXLEN, EMBED), jnp.float32),
    mesh=plsc.VectorSubcoreMesh(core_axis_name="c", subcore_axis_name="s"),
    compiler_params=pltpu.CompilerParams(use_tc_tiling_on_sc=False),
    scratch_types=[
        pltpu.VMEM((B_PER_W * GATHERS_PER_ROW, IDX_MINOR), jnp.int32),
        pltpu.VMEM((MAXLEN, EMBED), jnp.float32),
        pltpu.VMEM((NBUF // 2, 2 * MAXLEN, EMBED), jnp.float32),
        pltpu.SemaphoreType.DMA,
        pltpu.SemaphoreType.DMA,
    ],
)
def _embed_kernel(x_hbm, tok_hbm, pos_hbm, out_hbm, idx_v, pos_v, buf_v,
                  gsem, ssem):
    wid = lax.axis_index("s") * NC + lax.axis_index("c")
    pltpu.sync_copy(x_hbm.at[pl.ds(wid * B_PER_W * GATHERS_PER_ROW,
                                   B_PER_W * GATHERS_PER_ROW)], idx_v)

    def start_gather(b):
        k = b % NBUF
        return [
            pltpu.async_copy(
                tok_hbm.at[idx_v.at[GATHERS_PER_ROW * b + j]],
                buf_v.at[k // 2, pl.ds((k % 2) * MAXLEN + j * IDX_MINOR,
                                       IDX_MINOR)], gsem)
            for j in range(GATHERS_PER_ROW)
        ]

    gcp, scp = {}, {}
    for b in range(LOOKAHEAD):
        gcp[b] = start_gather(b)
    pltpu.sync_copy(pos_hbm, pos_v)
    for b in range(B_PER_W):
        nb = b + LOOKAHEAD
        if nb < B_PER_W:
            ob = nb - NBUF
            s = ob | 1
            if ob >= 0 and s in scp:
                scp.pop(s).wait()
            gcp[nb] = start_gather(nb)
        for c in gcp.pop(b):
            c.wait()
        k = b % NBUF

        def add_body(r, _, k=k):
            for c4 in range(EMBED // 16):
                sl = pl.ds(c4 * 16, 16)
                plsc.addupdate(buf_v.at[k // 2, (k % 2) * MAXLEN + r, sl],
                               pos_v[r, sl])
            return 0

        lax.fori_loop(0, MAXLEN, add_body, 0, unroll=4)
        if b % 2 == 1:
            scp[b] = pltpu.async_copy(
                buf_v.at[k // 2],
                out_hbm.at[pl.ds((wid * B_PER_W + b - 1) * MAXLEN,
                                 2 * MAXLEN)], ssem)
    for b in sorted(scp):
        scp[b].wait()


def kernel(x, token_table, pos_table):
    x2 = x.astype(jnp.int32).reshape(BATCH * MAXLEN // IDX_MINOR, IDX_MINOR)
    out = _embed_kernel(x2, token_table, pos_table)
    return out.reshape(BATCH, MAXLEN, EMBED)
